# baseline (speedup 1.0000x reference)
import jax
import jax.numpy as jnp
from jax.experimental import pallas as pl
from jax.experimental.pallas import tpu as pltpu

_TOP_K = 2
_LOAD_BALANCE_ALPHA = 0.01
_Z_LOSS_ALPHA = 0.0001


def _gate_tile(x_ref, wt_ref, bias_ref, rtemp_ref, ts_ref, ti_ref,
               part_ref):
    logits = jnp.dot(x_ref[...], wt_ref[...],
                     preferred_element_type=jnp.float32)
    lt = (logits.T + bias_ref[...]) * rtemp_ref[0, 0]
    c = jnp.max(lt)
    ex = jnp.exp(lt - c)
    se = jnp.sum(ex, axis=0, keepdims=True)
    num_e = ex.shape[0]
    eidx = jax.lax.broadcasted_iota(jnp.int32, ex.shape, 0)
    v1 = jnp.max(ex, axis=0, keepdims=True)
    i1 = jnp.min(jnp.where(ex == v1, eidx, num_e), axis=0, keepdims=True)
    h1 = eidx == i1
    exm = jnp.where(h1, -1.0, ex)
    v2 = jnp.max(exm, axis=0, keepdims=True)
    i2 = jnp.min(jnp.where(exm == v2, eidx, num_e), axis=0, keepdims=True)
    h2 = eidx == i2
    rden = 1.0 / (v1 + v2)
    ts_ref[...] = jnp.concatenate([v1 * rden, v2 * rden], axis=0)
    ti_ref[...] = jnp.concatenate([i1, i2], axis=0)
    hits = h1.astype(jnp.float32) + h2.astype(jnp.float32)
    lse = c + jnp.log(se)
    pf = jnp.sum(hits, axis=1, keepdims=True).T
    pp = jnp.sum(ex * (1.0 / se), axis=1, keepdims=True).T
    pz = jnp.broadcast_to(jnp.sum(lse * lse), pf.shape)
    part_ref[...] = jnp.concatenate([pf, pp, pz], axis=0)[None]


def kernel(x, W, expert_bias, temperature):
    B, S, H = x.shape
    E = W.shape[0]
    T = B * S
    x_flat = x.reshape(T, H)
    rtemp = (1.0 / jnp.asarray(temperature, jnp.float32)).reshape(1, 1)
    wt = W.T
    bias = expert_bias.reshape(E, 1)
    TT = 4096
    G = T // TT
    ts, ti, part = pl.pallas_call(
        _gate_tile,
        grid=(G,),
        in_specs=[
            pl.BlockSpec((TT, H), lambda i: (i, 0)),
            pl.BlockSpec((H, E), lambda i: (0, 0)),
            pl.BlockSpec((E, 1), lambda i: (0, 0)),
            pl.BlockSpec((1, 1), lambda i: (0, 0)),
        ],
        out_specs=[
            pl.BlockSpec((_TOP_K, TT), lambda i: (0, i)),
            pl.BlockSpec((_TOP_K, TT), lambda i: (0, i)),
            pl.BlockSpec((1, 3, E), lambda i: (i, 0, 0)),
        ],
        out_shape=[
            jax.ShapeDtypeStruct((_TOP_K, T), jnp.float32),
            jax.ShapeDtypeStruct((_TOP_K, T), jnp.int32),
            jax.ShapeDtypeStruct((G, 3, E), jnp.float32),
        ],
        compiler_params=pltpu.CompilerParams(
            dimension_semantics=("parallel",)),
    )(x_flat, wt, bias, rtemp)
    ts = ts.T
    ti = ti.T
    f = jnp.sum(part[:, 0, :], axis=0) / T
    P = jnp.sum(part[:, 1, :], axis=0) / T
    z = jnp.sum(part[:, 2, 0]) / T
    aux = _LOAD_BALANCE_ALPHA * E * jnp.sum(f * P)
    total = aux + _Z_LOSS_ALPHA * z
    return ts, ti, total
